# baseline (speedup 1.0000x reference)
import functools

import jax
import jax.numpy as jnp
from jax import lax
from jax.experimental import pallas as pl
from jax.experimental.pallas import tpu as pltpu
from jax.experimental.pallas import tpu_sc as plsc

DIM = 2048
HID = 2816
E = 8
K = 2
T = 4096

BLK = 512
P = 12288
NB = P // BLK
BH = 256
NH = HID // BH

NC = 2
NS = 16
NW = NC * NS
RPW = P // NW
CHG = 8
GBUF = 6
GLOOK = 4
TPW = T // NW
CH2 = 8

@functools.cache
def _sc_mesh():
    return plsc.VectorSubcoreMesh(
        core_axis_name="c", subcore_axis_name="s", num_cores=NC, num_subcores=NS
    )


def _router_body(x_ref, rw_ref, w_ref, i_ref):
    logits = jnp.dot(x_ref[...], rw_ref[...], preferred_element_type=jnp.float32)
    lane = lax.broadcasted_iota(jnp.int32, logits.shape, 1)
    logits = jnp.where(lane < E, logits, jnp.float32(-1e30))
    m = logits - jnp.max(logits, axis=1, keepdims=True)
    ex = jnp.exp(m)
    sm = ex / jnp.sum(ex, axis=1, keepdims=True)
    m1 = jnp.max(sm, axis=1, keepdims=True)
    i1 = jnp.min(jnp.where(sm == m1, lane, E), axis=1, keepdims=True)
    sm2 = jnp.where(lane == i1, jnp.float32(-1.0), sm)
    m2 = jnp.max(sm2, axis=1, keepdims=True)
    i2 = jnp.min(jnp.where(sm2 == m2, lane, E), axis=1, keepdims=True)
    zf = jnp.zeros_like(sm)
    w_ref[...] = jnp.where(lane == 0, m1, jnp.where(lane == 1, m2, zf))
    zi = jnp.zeros_like(lane)
    i_ref[...] = jnp.where(lane == 0, i1, jnp.where(lane == 1, i2, zi))


def _router(xf, router_w):
    BT = 512
    rwt = jnp.zeros((DIM, 128), jnp.float32).at[:, :E].set(router_w.T)
    w_out, i_out = pl.pallas_call(
        _router_body,
        grid=(T // BT,),
        in_specs=[
            pl.BlockSpec((BT, DIM), lambda i: (i, 0)),
            pl.BlockSpec((DIM, 128), lambda i: (0, 0)),
        ],
        out_specs=[
            pl.BlockSpec((BT, 128), lambda i: (i, 0)),
            pl.BlockSpec((BT, 128), lambda i: (i, 0)),
        ],
        out_shape=[
            jax.ShapeDtypeStruct((T, 128), jnp.float32),
            jax.ShapeDtypeStruct((T, 128), jnp.int32),
        ],
    )(xf, rwt)
    return w_out[:, :K], i_out[:, :K]


def _dispatch_maps(idx_top, w_top):
    flat_e = idx_top.reshape(-1)
    onehot = (flat_e[:, None] == jnp.arange(E, dtype=jnp.int32)[None, :])
    onehot = onehot.astype(jnp.int32)
    g = jnp.sum(onehot, axis=0)
    rank_mat = jnp.cumsum(onehot, axis=0) - onehot
    rank = jnp.take_along_axis(rank_mat, flat_e[:, None], axis=1)[:, 0]
    gp = ((g + BLK - 1) // BLK) * BLK
    cum = jnp.cumsum(gp)
    po = cum - gp
    pos = (po[flat_e] + rank).astype(jnp.int32)
    tok = (jnp.arange(T * K, dtype=jnp.int32) // K)
    tok_map = jnp.zeros((P,), jnp.int32).at[pos].set(tok)
    wgt_map = jnp.zeros((P,), jnp.float32).at[pos].set(w_top.reshape(-1))
    starts = jnp.arange(NB, dtype=jnp.int32) * BLK
    block_e = jnp.sum(starts[:, None] >= cum[None, :], axis=1).astype(jnp.int32)
    block_e = jnp.minimum(block_e, E - 1)
    pos_kt = pos.reshape(T, K).T
    return tok_map, wgt_map, block_e, pos_kt


def _gather_body(xf_hbm, tok_hbm, out_hbm, idx_v, *bufs_and_sems):
    bufs = bufs_and_sems[:GBUF]
    sgs = bufs_and_sems[GBUF:2 * GBUF]
    sss = bufs_and_sems[2 * GBUF:3 * GBUF]
    wid = lax.axis_index("s") * NC + lax.axis_index("c")
    base = wid * RPW
    pltpu.sync_copy(tok_hbm.at[pl.ds(base, RPW)], idx_v)
    n = RPW // CHG
    gh = [None] * GBUF
    sh = [None] * GBUF

    def start_gather(c):
        p = c % GBUF
        if sh[p] is not None:
            sh[p].wait()
            sh[p] = None
        gh[p] = pltpu.async_copy(
            xf_hbm.at[idx_v.at[pl.ds(c * CHG, CHG)]], bufs[p], sgs[p]
        )

    for c in range(min(GLOOK, n)):
        start_gather(c)
    for c in range(n):
        p = c % GBUF
        if c + GLOOK < n:
            start_gather(c + GLOOK)
        gh[p].wait()
        sh[p] = pltpu.async_copy(
            bufs[p], out_hbm.at[pl.ds(base + c * CHG, CHG)], sss[p]
        )
    for p in range(GBUF):
        if sh[p] is not None:
            sh[p].wait()


@functools.cache
def _gather():
    return pl.kernel(
        _gather_body,
        out_type=jax.ShapeDtypeStruct((P, DIM), jnp.float32),
        mesh=_sc_mesh(),
        scratch_types=(
            [pltpu.VMEM((RPW,), jnp.int32)]
            + [pltpu.VMEM((CHG, DIM), jnp.float32)] * GBUF
            + [pltpu.SemaphoreType.DMA] * (2 * GBUF)
        ),
    )


def _gmm_body(e_map, wgt_ref, xg_ref, w1_ref, w3_ref, w2_ref, out_ref):
    j = pl.program_id(1)
    xb = xg_ref[...]
    a = jnp.dot(xb, w1_ref[0], preferred_element_type=jnp.float32)
    b = jnp.dot(xb, w3_ref[0], preferred_element_type=jnp.float32)
    h = ((a * jax.nn.sigmoid(a)) * b).astype(jnp.bfloat16)
    contrib = jnp.dot(h, w2_ref[0], preferred_element_type=jnp.float32)
    contrib = contrib * wgt_ref[0, 0, :][:, None]

    @pl.when(j == 0)
    def _():
        out_ref[...] = contrib

    @pl.when(j > 0)
    def _():
        out_ref[...] += contrib


def _gmm(block_e, wgt_map, xg, w1, w2, w3):
    wgt3 = wgt_map.reshape(NB, 1, BLK)
    grid_spec = pltpu.PrefetchScalarGridSpec(
        num_scalar_prefetch=1,
        grid=(NB, NH),
        in_specs=[
            pl.BlockSpec((1, 1, BLK), lambda i, j, e: (i, 0, 0)),
            pl.BlockSpec((BLK, DIM), lambda i, j, e: (i, 0)),
            pl.BlockSpec((1, DIM, BH), lambda i, j, e: (e[i], 0, j)),
            pl.BlockSpec((1, DIM, BH), lambda i, j, e: (e[i], 0, j)),
            pl.BlockSpec((1, BH, DIM), lambda i, j, e: (e[i], j, 0)),
        ],
        out_specs=pl.BlockSpec((BLK, DIM), lambda i, j, e: (i, 0)),
    )
    return pl.pallas_call(
        _gmm_body,
        grid_spec=grid_spec,
        out_shape=jax.ShapeDtypeStruct((P, DIM), jnp.float32),
        compiler_params=pltpu.CompilerParams(
            dimension_semantics=("arbitrary", "arbitrary"),
        ),
    )(block_e, wgt3, xg, w1, w3, w2)


def _combine_body(
    y_hbm, pos_hbm, out_hbm,
    i0, i1,
    r0a, r1a, oba, r0b, r1b, obb,
    g0a, g1a, g0b, g1b, sa, sb,
):
    wid = lax.axis_index("s") * NC + lax.axis_index("c")
    tb = wid * TPW
    pltpu.sync_copy(pos_hbm.at[0, pl.ds(tb, TPW)], i0)
    pltpu.sync_copy(pos_hbm.at[1, pl.ds(tb, TPW)], i1)
    r0s = (r0a, r0b)
    r1s = (r1a, r1b)
    obs = (oba, obb)
    g0s = (g0a, g0b)
    g1s = (g1a, g1b)
    sss = (sa, sb)
    n = TPW // CH2
    gh = [None, None]
    sh = [None, None]
    gh[0] = (
        pltpu.async_copy(y_hbm.at[i0.at[pl.ds(0, CH2)]], r0a, g0a),
        pltpu.async_copy(y_hbm.at[i1.at[pl.ds(0, CH2)]], r1a, g1a),
    )
    for c in range(n):
        p = c % 2
        q = (c + 1) % 2
        if c + 1 < n:
            gh[q] = (
                pltpu.async_copy(
                    y_hbm.at[i0.at[pl.ds((c + 1) * CH2, CH2)]], r0s[q], g0s[q]
                ),
                pltpu.async_copy(
                    y_hbm.at[i1.at[pl.ds((c + 1) * CH2, CH2)]], r1s[q], g1s[q]
                ),
            )
        gh[p][0].wait()
        gh[p][1].wait()
        if sh[p] is not None:
            sh[p].wait()
        r0 = r0s[p]
        r1 = r1s[p]
        ob = obs[p]

        @plsc.parallel_loop(0, CH2 * DIM // 16, 1, unroll=8)
        def _add(i):
            r = i // (DIM // 16)
            col = (i % (DIM // 16)) * 16
            ob[r, pl.ds(col, 16)] = r0[r, pl.ds(col, 16)] + r1[r, pl.ds(col, 16)]

        sh[p] = pltpu.async_copy(
            ob, out_hbm.at[pl.ds(tb + c * CH2, CH2)], sss[p]
        )
    for p in range(2):
        if sh[p] is not None:
            sh[p].wait()


@functools.cache
def _combine():
    return pl.kernel(
        _combine_body,
        out_type=jax.ShapeDtypeStruct((T, DIM), jnp.float32),
        mesh=_sc_mesh(),
        scratch_types=[
            pltpu.VMEM((TPW,), jnp.int32),
            pltpu.VMEM((TPW,), jnp.int32),
            pltpu.VMEM((CH2, DIM), jnp.float32),
            pltpu.VMEM((CH2, DIM), jnp.float32),
            pltpu.VMEM((CH2, DIM), jnp.float32),
            pltpu.VMEM((CH2, DIM), jnp.float32),
            pltpu.VMEM((CH2, DIM), jnp.float32),
            pltpu.VMEM((CH2, DIM), jnp.float32),
            pltpu.SemaphoreType.DMA,
            pltpu.SemaphoreType.DMA,
            pltpu.SemaphoreType.DMA,
            pltpu.SemaphoreType.DMA,
            pltpu.SemaphoreType.DMA,
            pltpu.SemaphoreType.DMA,
        ],
    )


def kernel(x, router_w, w1, w2, w3):
    bsz, seqlen, dim = x.shape
    xf = x.reshape(-1, dim)
    w_top, idx_top = _router(xf, router_w)
    tok_map, wgt_map, block_e, pos_kt = _dispatch_maps(idx_top, w_top)
    xg = _gather()(xf, tok_map)
    y = _gmm(block_e, wgt_map, xg, w1, w2, w3)
    out = _combine()(y, pos_kt)
    return out.reshape(bsz, seqlen, dim)


# speedup vs baseline: 1.0071x; 1.0071x over previous
import functools

import jax
import jax.numpy as jnp
from jax import lax
from jax.experimental import pallas as pl
from jax.experimental.pallas import tpu as pltpu
from jax.experimental.pallas import tpu_sc as plsc

DIM = 2048
HID = 2816
E = 8
K = 2
T = 4096

BLK = 512
P = 12288
NB = P // BLK
BH = 256
NH = HID // BH

NC = 2
NS = 16
NW = NC * NS
RPW = P // NW
CHG = 16
GBUF = 3
GLOOK = 2
TPW = T // NW
CH2 = 8

@functools.cache
def _sc_mesh():
    return plsc.VectorSubcoreMesh(
        core_axis_name="c", subcore_axis_name="s", num_cores=NC, num_subcores=NS
    )


def _router_body(x_ref, rw_ref, w_ref, i_ref):
    logits = jnp.dot(x_ref[...], rw_ref[...], preferred_element_type=jnp.float32)
    lane = lax.broadcasted_iota(jnp.int32, logits.shape, 1)
    logits = jnp.where(lane < E, logits, jnp.float32(-1e30))
    m = logits - jnp.max(logits, axis=1, keepdims=True)
    ex = jnp.exp(m)
    sm = ex / jnp.sum(ex, axis=1, keepdims=True)
    m1 = jnp.max(sm, axis=1, keepdims=True)
    i1 = jnp.min(jnp.where(sm == m1, lane, E), axis=1, keepdims=True)
    sm2 = jnp.where(lane == i1, jnp.float32(-1.0), sm)
    m2 = jnp.max(sm2, axis=1, keepdims=True)
    i2 = jnp.min(jnp.where(sm2 == m2, lane, E), axis=1, keepdims=True)
    zf = jnp.zeros_like(sm)
    w_ref[...] = jnp.where(lane == 0, m1, jnp.where(lane == 1, m2, zf))
    zi = jnp.zeros_like(lane)
    i_ref[...] = jnp.where(lane == 0, i1, jnp.where(lane == 1, i2, zi))


def _router(xf, router_w):
    BT = 512
    rwt = jnp.zeros((DIM, 128), jnp.float32).at[:, :E].set(router_w.T)
    w_out, i_out = pl.pallas_call(
        _router_body,
        grid=(T // BT,),
        in_specs=[
            pl.BlockSpec((BT, DIM), lambda i: (i, 0)),
            pl.BlockSpec((DIM, 128), lambda i: (0, 0)),
        ],
        out_specs=[
            pl.BlockSpec((BT, 128), lambda i: (i, 0)),
            pl.BlockSpec((BT, 128), lambda i: (i, 0)),
        ],
        out_shape=[
            jax.ShapeDtypeStruct((T, 128), jnp.float32),
            jax.ShapeDtypeStruct((T, 128), jnp.int32),
        ],
    )(xf, rwt)
    return w_out[:, :K], i_out[:, :K]


def _dispatch_maps(idx_top, w_top):
    flat_e = idx_top.reshape(-1)
    onehot = (flat_e[:, None] == jnp.arange(E, dtype=jnp.int32)[None, :])
    onehot = onehot.astype(jnp.int32)
    g = jnp.sum(onehot, axis=0)
    rank_mat = jnp.cumsum(onehot, axis=0) - onehot
    rank = jnp.take_along_axis(rank_mat, flat_e[:, None], axis=1)[:, 0]
    gp = ((g + BLK - 1) // BLK) * BLK
    cum = jnp.cumsum(gp)
    po = cum - gp
    pos = (po[flat_e] + rank).astype(jnp.int32)
    tok = (jnp.arange(T * K, dtype=jnp.int32) // K)
    tok_map = jnp.zeros((P,), jnp.int32).at[pos].set(tok)
    wgt_map = jnp.zeros((P,), jnp.float32).at[pos].set(w_top.reshape(-1))
    starts = jnp.arange(NB, dtype=jnp.int32) * BLK
    block_e = jnp.sum(starts[:, None] >= cum[None, :], axis=1).astype(jnp.int32)
    block_e = jnp.minimum(block_e, E - 1)
    pos_kt = pos.reshape(T, K).T
    return tok_map, wgt_map, block_e, pos_kt


def _gather_body(xf_hbm, tok_hbm, out_hbm, idx_v, *bufs_and_sems):
    bufs = bufs_and_sems[:GBUF]
    sgs = bufs_and_sems[GBUF:2 * GBUF]
    sss = bufs_and_sems[2 * GBUF:3 * GBUF]
    wid = lax.axis_index("s") * NC + lax.axis_index("c")
    base = wid * RPW
    pltpu.sync_copy(tok_hbm.at[pl.ds(base, RPW)], idx_v)
    n = RPW // CHG
    gh = [None] * GBUF
    sh = [None] * GBUF

    def start_gather(c):
        p = c % GBUF
        if sh[p] is not None:
            sh[p].wait()
            sh[p] = None
        gh[p] = pltpu.async_copy(
            xf_hbm.at[idx_v.at[pl.ds(c * CHG, CHG)]], bufs[p], sgs[p]
        )

    for c in range(min(GLOOK, n)):
        start_gather(c)
    for c in range(n):
        p = c % GBUF
        if c + GLOOK < n:
            start_gather(c + GLOOK)
        gh[p].wait()
        sh[p] = pltpu.async_copy(
            bufs[p], out_hbm.at[pl.ds(base + c * CHG, CHG)], sss[p]
        )
    for p in range(GBUF):
        if sh[p] is not None:
            sh[p].wait()


@functools.cache
def _gather():
    return pl.kernel(
        _gather_body,
        out_type=jax.ShapeDtypeStruct((P, DIM), jnp.float32),
        mesh=_sc_mesh(),
        scratch_types=(
            [pltpu.VMEM((RPW,), jnp.int32)]
            + [pltpu.VMEM((CHG, DIM), jnp.float32)] * GBUF
            + [pltpu.SemaphoreType.DMA] * (2 * GBUF)
        ),
    )


def _gmm_body(e_map, wgt_ref, xg_ref, w1_ref, w3_ref, w2_ref, out_ref):
    j = pl.program_id(1)
    xb = xg_ref[...]
    a = jnp.dot(xb, w1_ref[0], preferred_element_type=jnp.float32)
    b = jnp.dot(xb, w3_ref[0], preferred_element_type=jnp.float32)
    h = ((a * jax.nn.sigmoid(a)) * b).astype(jnp.bfloat16)
    contrib = jnp.dot(h, w2_ref[0], preferred_element_type=jnp.float32)
    contrib = contrib * wgt_ref[0, 0, :][:, None]

    @pl.when(j == 0)
    def _():
        out_ref[...] = contrib

    @pl.when(j > 0)
    def _():
        out_ref[...] += contrib


def _gmm(block_e, wgt_map, xg, w1, w2, w3):
    wgt3 = wgt_map.reshape(NB, 1, BLK)
    grid_spec = pltpu.PrefetchScalarGridSpec(
        num_scalar_prefetch=1,
        grid=(NB, NH),
        in_specs=[
            pl.BlockSpec((1, 1, BLK), lambda i, j, e: (i, 0, 0)),
            pl.BlockSpec((BLK, DIM), lambda i, j, e: (i, 0)),
            pl.BlockSpec((1, DIM, BH), lambda i, j, e: (e[i], 0, j)),
            pl.BlockSpec((1, DIM, BH), lambda i, j, e: (e[i], 0, j)),
            pl.BlockSpec((1, BH, DIM), lambda i, j, e: (e[i], j, 0)),
        ],
        out_specs=pl.BlockSpec((BLK, DIM), lambda i, j, e: (i, 0)),
    )
    return pl.pallas_call(
        _gmm_body,
        grid_spec=grid_spec,
        out_shape=jax.ShapeDtypeStruct((P, DIM), jnp.float32),
        compiler_params=pltpu.CompilerParams(
            dimension_semantics=("arbitrary", "arbitrary"),
        ),
    )(block_e, wgt3, xg, w1, w3, w2)


def _combine_body(
    y_hbm, pos_hbm, out_hbm,
    i0, i1,
    r0a, r1a, oba, r0b, r1b, obb,
    g0a, g1a, g0b, g1b, sa, sb,
):
    wid = lax.axis_index("s") * NC + lax.axis_index("c")
    tb = wid * TPW
    pltpu.sync_copy(pos_hbm.at[0, pl.ds(tb, TPW)], i0)
    pltpu.sync_copy(pos_hbm.at[1, pl.ds(tb, TPW)], i1)
    r0s = (r0a, r0b)
    r1s = (r1a, r1b)
    obs = (oba, obb)
    g0s = (g0a, g0b)
    g1s = (g1a, g1b)
    sss = (sa, sb)
    n = TPW // CH2
    gh = [None, None]
    sh = [None, None]
    gh[0] = (
        pltpu.async_copy(y_hbm.at[i0.at[pl.ds(0, CH2)]], r0a, g0a),
        pltpu.async_copy(y_hbm.at[i1.at[pl.ds(0, CH2)]], r1a, g1a),
    )
    for c in range(n):
        p = c % 2
        q = (c + 1) % 2
        if c + 1 < n:
            gh[q] = (
                pltpu.async_copy(
                    y_hbm.at[i0.at[pl.ds((c + 1) * CH2, CH2)]], r0s[q], g0s[q]
                ),
                pltpu.async_copy(
                    y_hbm.at[i1.at[pl.ds((c + 1) * CH2, CH2)]], r1s[q], g1s[q]
                ),
            )
        gh[p][0].wait()
        gh[p][1].wait()
        if sh[p] is not None:
            sh[p].wait()
        r0 = r0s[p]
        r1 = r1s[p]
        ob = obs[p]

        @plsc.parallel_loop(0, CH2 * DIM // 16, 1, unroll=8)
        def _add(i):
            r = i // (DIM // 16)
            col = (i % (DIM // 16)) * 16
            ob[r, pl.ds(col, 16)] = r0[r, pl.ds(col, 16)] + r1[r, pl.ds(col, 16)]

        sh[p] = pltpu.async_copy(
            ob, out_hbm.at[pl.ds(tb + c * CH2, CH2)], sss[p]
        )
    for p in range(2):
        if sh[p] is not None:
            sh[p].wait()


@functools.cache
def _combine():
    return pl.kernel(
        _combine_body,
        out_type=jax.ShapeDtypeStruct((T, DIM), jnp.float32),
        mesh=_sc_mesh(),
        scratch_types=[
            pltpu.VMEM((TPW,), jnp.int32),
            pltpu.VMEM((TPW,), jnp.int32),
            pltpu.VMEM((CH2, DIM), jnp.float32),
            pltpu.VMEM((CH2, DIM), jnp.float32),
            pltpu.VMEM((CH2, DIM), jnp.float32),
            pltpu.VMEM((CH2, DIM), jnp.float32),
            pltpu.VMEM((CH2, DIM), jnp.float32),
            pltpu.VMEM((CH2, DIM), jnp.float32),
            pltpu.SemaphoreType.DMA,
            pltpu.SemaphoreType.DMA,
            pltpu.SemaphoreType.DMA,
            pltpu.SemaphoreType.DMA,
            pltpu.SemaphoreType.DMA,
            pltpu.SemaphoreType.DMA,
        ],
    )


def kernel(x, router_w, w1, w2, w3):
    bsz, seqlen, dim = x.shape
    xf = x.reshape(-1, dim)
    w_top, idx_top = _router(xf, router_w)
    tok_map, wgt_map, block_e, pos_kt = _dispatch_maps(idx_top, w_top)
    xg = _gather()(xf, tok_map)
    y = _gmm(block_e, wgt_map, xg, w1, w2, w3)
    out = _combine()(y, pos_kt)
    return out.reshape(bsz, seqlen, dim)


# speedup vs baseline: 1.1477x; 1.1396x over previous
import functools

import jax
import jax.numpy as jnp
from jax import lax
from jax.experimental import pallas as pl
from jax.experimental.pallas import tpu as pltpu
from jax.experimental.pallas import tpu_sc as plsc

DIM = 2048
HID = 2816
E = 8
K = 2
T = 4096

BLK = 512
P = 12288
NB = P // BLK
BH = 256
NH = HID // BH

NC = 2
NS = 16
NW = NC * NS
RPW = P // NW
CHG = 16
GBUF = 3
GLOOK = 2
TPW = T // NW
CH2 = 8

@functools.cache
def _sc_mesh():
    return plsc.VectorSubcoreMesh(
        core_axis_name="c", subcore_axis_name="s", num_cores=NC, num_subcores=NS
    )


def _router_body(x_ref, rw_ref, w_ref, i_ref):
    logits = jnp.dot(x_ref[...], rw_ref[...], preferred_element_type=jnp.float32)
    lane = lax.broadcasted_iota(jnp.int32, logits.shape, 1)
    logits = jnp.where(lane < E, logits, jnp.float32(-1e30))
    m = logits - jnp.max(logits, axis=1, keepdims=True)
    ex = jnp.exp(m)
    sm = ex / jnp.sum(ex, axis=1, keepdims=True)
    m1 = jnp.max(sm, axis=1, keepdims=True)
    i1 = jnp.min(jnp.where(sm == m1, lane, E), axis=1, keepdims=True)
    sm2 = jnp.where(lane == i1, jnp.float32(-1.0), sm)
    m2 = jnp.max(sm2, axis=1, keepdims=True)
    i2 = jnp.min(jnp.where(sm2 == m2, lane, E), axis=1, keepdims=True)
    zf = jnp.zeros_like(sm)
    w_ref[...] = jnp.where(lane == 0, m1, jnp.where(lane == 1, m2, zf))
    zi = jnp.zeros_like(lane)
    i_ref[...] = jnp.where(lane == 0, i1, jnp.where(lane == 1, i2, zi))


def _router(xf, router_w):
    BT = 512
    rwt = jnp.zeros((DIM, 128), jnp.float32).at[:, :E].set(router_w.T)
    w_out, i_out = pl.pallas_call(
        _router_body,
        grid=(T // BT,),
        in_specs=[
            pl.BlockSpec((BT, DIM), lambda i: (i, 0)),
            pl.BlockSpec((DIM, 128), lambda i: (0, 0)),
        ],
        out_specs=[
            pl.BlockSpec((BT, 128), lambda i: (i, 0)),
            pl.BlockSpec((BT, 128), lambda i: (i, 0)),
        ],
        out_shape=[
            jax.ShapeDtypeStruct((T, 128), jnp.float32),
            jax.ShapeDtypeStruct((T, 128), jnp.int32),
        ],
    )(xf, rwt)
    return w_out[:, :K], i_out[:, :K]


def _dispatch_maps(idx_top, w_top):
    flat_e = idx_top.reshape(-1)
    onehot = (flat_e[:, None] == jnp.arange(E, dtype=jnp.int32)[None, :])
    onehot = onehot.astype(jnp.int32)
    g = jnp.sum(onehot, axis=0)
    rank_mat = jnp.cumsum(onehot, axis=0) - onehot
    rank = jnp.take_along_axis(rank_mat, flat_e[:, None], axis=1)[:, 0]
    gp = ((g + BLK - 1) // BLK) * BLK
    cum = jnp.cumsum(gp)
    po = cum - gp
    pos = (po[flat_e] + rank).astype(jnp.int32)
    tok = (jnp.arange(T * K, dtype=jnp.int32) // K)
    tok_map = jnp.zeros((P,), jnp.int32).at[pos].set(tok)
    wgt_map = jnp.zeros((P,), jnp.float32).at[pos].set(w_top.reshape(-1))
    starts = jnp.arange(NB, dtype=jnp.int32) * BLK
    block_e = jnp.sum(starts[:, None] >= cum[None, :], axis=1).astype(jnp.int32)
    block_e = jnp.minimum(block_e, E - 1)
    nbu = cum[-1] // BLK
    s_ar = jnp.arange(NB * NH, dtype=jnp.int32)
    active = s_ar < nbu * NH
    im = jnp.where(active, s_ar // NH, nbu - 1)
    jm = jnp.where(active, s_ar % NH, NH - 1)
    em = block_e[im]
    mp = jnp.stack([im, jm, em, active.astype(jnp.int32)])
    pos_kt = pos.reshape(T, K).T
    return tok_map, wgt_map, mp, pos_kt


def _gather_body(xf_hbm, tok_hbm, out_hbm, idx_v, *bufs_and_sems):
    bufs = bufs_and_sems[:GBUF]
    sgs = bufs_and_sems[GBUF:2 * GBUF]
    sss = bufs_and_sems[2 * GBUF:3 * GBUF]
    wid = lax.axis_index("s") * NC + lax.axis_index("c")
    base = wid * RPW
    pltpu.sync_copy(tok_hbm.at[pl.ds(base, RPW)], idx_v)
    n = RPW // CHG
    gh = [None] * GBUF
    sh = [None] * GBUF

    def start_gather(c):
        p = c % GBUF
        if sh[p] is not None:
            sh[p].wait()
            sh[p] = None
        gh[p] = pltpu.async_copy(
            xf_hbm.at[idx_v.at[pl.ds(c * CHG, CHG)]], bufs[p], sgs[p]
        )

    for c in range(min(GLOOK, n)):
        start_gather(c)
    for c in range(n):
        p = c % GBUF
        if c + GLOOK < n:
            start_gather(c + GLOOK)
        gh[p].wait()
        sh[p] = pltpu.async_copy(
            bufs[p], out_hbm.at[pl.ds(base + c * CHG, CHG)], sss[p]
        )
    for p in range(GBUF):
        if sh[p] is not None:
            sh[p].wait()


@functools.cache
def _gather():
    return pl.kernel(
        _gather_body,
        out_type=jax.ShapeDtypeStruct((P, DIM), jnp.float32),
        mesh=_sc_mesh(),
        scratch_types=(
            [pltpu.VMEM((RPW,), jnp.int32)]
            + [pltpu.VMEM((CHG, DIM), jnp.float32)] * GBUF
            + [pltpu.SemaphoreType.DMA] * (2 * GBUF)
        ),
    )


def _gmm_body(mp_ref, wgt_ref, xg_ref, w1_ref, w3_ref, w2_ref, out_ref):
    s = pl.program_id(0)
    jv = mp_ref[1, s]
    act = mp_ref[3, s] > 0

    @pl.when(act)
    def _():
        xb = xg_ref[...]
        a = jnp.dot(xb, w1_ref[0], preferred_element_type=jnp.float32)
        b = jnp.dot(xb, w3_ref[0], preferred_element_type=jnp.float32)
        h = ((a * jax.nn.sigmoid(a)) * b).astype(jnp.bfloat16)
        contrib = jnp.dot(h, w2_ref[0], preferred_element_type=jnp.float32)
        contrib = contrib * wgt_ref[0, 0, :][:, None]

        @pl.when(jv == 0)
        def _():
            out_ref[...] = contrib

        @pl.when(jv > 0)
        def _():
            out_ref[...] += contrib


def _gmm(mp, wgt_map, xg, w1, w2, w3):
    wgt3 = wgt_map.reshape(NB, 1, BLK)
    grid_spec = pltpu.PrefetchScalarGridSpec(
        num_scalar_prefetch=1,
        grid=(NB * NH,),
        in_specs=[
            pl.BlockSpec((1, 1, BLK), lambda s, m: (m[0, s], 0, 0)),
            pl.BlockSpec((BLK, DIM), lambda s, m: (m[0, s], 0)),
            pl.BlockSpec((1, DIM, BH), lambda s, m: (m[2, s], 0, m[1, s])),
            pl.BlockSpec((1, DIM, BH), lambda s, m: (m[2, s], 0, m[1, s])),
            pl.BlockSpec((1, BH, DIM), lambda s, m: (m[2, s], m[1, s], 0)),
        ],
        out_specs=pl.BlockSpec((BLK, DIM), lambda s, m: (m[0, s], 0)),
    )
    return pl.pallas_call(
        _gmm_body,
        grid_spec=grid_spec,
        out_shape=jax.ShapeDtypeStruct((P, DIM), jnp.float32),
        compiler_params=pltpu.CompilerParams(
            dimension_semantics=("arbitrary",),
        ),
    )(mp, wgt3, xg, w1, w3, w2)


def _combine_body(
    y_hbm, pos_hbm, out_hbm,
    i0, i1,
    r0a, r1a, oba, r0b, r1b, obb,
    g0a, g1a, g0b, g1b, sa, sb,
):
    wid = lax.axis_index("s") * NC + lax.axis_index("c")
    tb = wid * TPW
    pltpu.sync_copy(pos_hbm.at[0, pl.ds(tb, TPW)], i0)
    pltpu.sync_copy(pos_hbm.at[1, pl.ds(tb, TPW)], i1)
    r0s = (r0a, r0b)
    r1s = (r1a, r1b)
    obs = (oba, obb)
    g0s = (g0a, g0b)
    g1s = (g1a, g1b)
    sss = (sa, sb)
    n = TPW // CH2
    gh = [None, None]
    sh = [None, None]
    gh[0] = (
        pltpu.async_copy(y_hbm.at[i0.at[pl.ds(0, CH2)]], r0a, g0a),
        pltpu.async_copy(y_hbm.at[i1.at[pl.ds(0, CH2)]], r1a, g1a),
    )
    for c in range(n):
        p = c % 2
        q = (c + 1) % 2
        if c + 1 < n:
            gh[q] = (
                pltpu.async_copy(
                    y_hbm.at[i0.at[pl.ds((c + 1) * CH2, CH2)]], r0s[q], g0s[q]
                ),
                pltpu.async_copy(
                    y_hbm.at[i1.at[pl.ds((c + 1) * CH2, CH2)]], r1s[q], g1s[q]
                ),
            )
        gh[p][0].wait()
        gh[p][1].wait()
        if sh[p] is not None:
            sh[p].wait()
        r0 = r0s[p]
        r1 = r1s[p]
        ob = obs[p]

        @plsc.parallel_loop(0, CH2 * DIM // 16, 1, unroll=8)
        def _add(i):
            r = i // (DIM // 16)
            col = (i % (DIM // 16)) * 16
            ob[r, pl.ds(col, 16)] = r0[r, pl.ds(col, 16)] + r1[r, pl.ds(col, 16)]

        sh[p] = pltpu.async_copy(
            ob, out_hbm.at[pl.ds(tb + c * CH2, CH2)], sss[p]
        )
    for p in range(2):
        if sh[p] is not None:
            sh[p].wait()


@functools.cache
def _combine():
    return pl.kernel(
        _combine_body,
        out_type=jax.ShapeDtypeStruct((T, DIM), jnp.float32),
        mesh=_sc_mesh(),
        scratch_types=[
            pltpu.VMEM((TPW,), jnp.int32),
            pltpu.VMEM((TPW,), jnp.int32),
            pltpu.VMEM((CH2, DIM), jnp.float32),
            pltpu.VMEM((CH2, DIM), jnp.float32),
            pltpu.VMEM((CH2, DIM), jnp.float32),
            pltpu.VMEM((CH2, DIM), jnp.float32),
            pltpu.VMEM((CH2, DIM), jnp.float32),
            pltpu.VMEM((CH2, DIM), jnp.float32),
            pltpu.SemaphoreType.DMA,
            pltpu.SemaphoreType.DMA,
            pltpu.SemaphoreType.DMA,
            pltpu.SemaphoreType.DMA,
            pltpu.SemaphoreType.DMA,
            pltpu.SemaphoreType.DMA,
        ],
    )


def kernel(x, router_w, w1, w2, w3):
    bsz, seqlen, dim = x.shape
    xf = x.reshape(-1, dim)
    w_top, idx_top = _router(xf, router_w)
    tok_map, wgt_map, mp, pos_kt = _dispatch_maps(idx_top, w_top)
    xg = _gather()(xf, tok_map)
    y = _gmm(mp, wgt_map, xg, w1, w2, w3)
    out = _combine()(y, pos_kt)
    return out.reshape(bsz, seqlen, dim)


# speedup vs baseline: 1.1526x; 1.0043x over previous
import functools

import jax
import jax.numpy as jnp
from jax import lax
from jax.experimental import pallas as pl
from jax.experimental.pallas import tpu as pltpu
from jax.experimental.pallas import tpu_sc as plsc

DIM = 2048
HID = 2816
E = 8
K = 2
T = 4096

BLK = 512
P = 12288
NB = P // BLK
BH = 256
NH = HID // BH

NC = 2
NS = 16
NW = NC * NS
RPW = P // NW
CHG = 24
GBUF = 2
GLOOK = 1
TPW = T // NW
CH2 = 8

@functools.cache
def _sc_mesh():
    return plsc.VectorSubcoreMesh(
        core_axis_name="c", subcore_axis_name="s", num_cores=NC, num_subcores=NS
    )


def _router_body(x_ref, rw_ref, w_ref, i_ref):
    logits = jnp.dot(x_ref[...], rw_ref[...], preferred_element_type=jnp.float32)
    lane = lax.broadcasted_iota(jnp.int32, logits.shape, 1)
    logits = jnp.where(lane < E, logits, jnp.float32(-1e30))
    m = logits - jnp.max(logits, axis=1, keepdims=True)
    ex = jnp.exp(m)
    sm = ex / jnp.sum(ex, axis=1, keepdims=True)
    m1 = jnp.max(sm, axis=1, keepdims=True)
    i1 = jnp.min(jnp.where(sm == m1, lane, E), axis=1, keepdims=True)
    sm2 = jnp.where(lane == i1, jnp.float32(-1.0), sm)
    m2 = jnp.max(sm2, axis=1, keepdims=True)
    i2 = jnp.min(jnp.where(sm2 == m2, lane, E), axis=1, keepdims=True)
    zf = jnp.zeros_like(sm)
    w_ref[...] = jnp.where(lane == 0, m1, jnp.where(lane == 1, m2, zf))
    zi = jnp.zeros_like(lane)
    i_ref[...] = jnp.where(lane == 0, i1, jnp.where(lane == 1, i2, zi))


def _router(xf, router_w):
    BT = 512
    rwt = jnp.zeros((DIM, 128), jnp.float32).at[:, :E].set(router_w.T)
    w_out, i_out = pl.pallas_call(
        _router_body,
        grid=(T // BT,),
        in_specs=[
            pl.BlockSpec((BT, DIM), lambda i: (i, 0)),
            pl.BlockSpec((DIM, 128), lambda i: (0, 0)),
        ],
        out_specs=[
            pl.BlockSpec((BT, 128), lambda i: (i, 0)),
            pl.BlockSpec((BT, 128), lambda i: (i, 0)),
        ],
        out_shape=[
            jax.ShapeDtypeStruct((T, 128), jnp.float32),
            jax.ShapeDtypeStruct((T, 128), jnp.int32),
        ],
    )(xf, rwt)
    return w_out[:, :K], i_out[:, :K]


def _dispatch_maps(idx_top, w_top):
    flat_e = idx_top.reshape(-1)
    onehot = (flat_e[:, None] == jnp.arange(E, dtype=jnp.int32)[None, :])
    onehot = onehot.astype(jnp.int32)
    g = jnp.sum(onehot, axis=0)
    rank_mat = jnp.cumsum(onehot, axis=0) - onehot
    rank = jnp.take_along_axis(rank_mat, flat_e[:, None], axis=1)[:, 0]
    gp = ((g + BLK - 1) // BLK) * BLK
    cum = jnp.cumsum(gp)
    po = cum - gp
    pos = (po[flat_e] + rank).astype(jnp.int32)
    tok = (jnp.arange(T * K, dtype=jnp.int32) // K)
    tok_map = jnp.zeros((P,), jnp.int32).at[pos].set(tok)
    wgt_map = jnp.zeros((P,), jnp.float32).at[pos].set(w_top.reshape(-1))
    starts = jnp.arange(NB, dtype=jnp.int32) * BLK
    block_e = jnp.sum(starts[:, None] >= cum[None, :], axis=1).astype(jnp.int32)
    block_e = jnp.minimum(block_e, E - 1)
    nbu = cum[-1] // BLK
    s_ar = jnp.arange(NB * NH, dtype=jnp.int32)
    active = s_ar < nbu * NH
    im = jnp.where(active, s_ar // NH, nbu - 1)
    jm = jnp.where(active, s_ar % NH, NH - 1)
    em = block_e[im]
    mp = jnp.stack([im, jm, em, active.astype(jnp.int32)])
    pos_kt = pos.reshape(T, K).T
    return tok_map, wgt_map, mp, pos_kt


def _gather_body(xf_hbm, tok_hbm, out_hbm, idx_v, *bufs_and_sems):
    bufs = bufs_and_sems[:GBUF]
    sgs = bufs_and_sems[GBUF:2 * GBUF]
    sss = bufs_and_sems[2 * GBUF:3 * GBUF]
    wid = lax.axis_index("s") * NC + lax.axis_index("c")
    base = wid * RPW
    pltpu.sync_copy(tok_hbm.at[pl.ds(base, RPW)], idx_v)
    n = RPW // CHG
    gh = [None] * GBUF
    sh = [None] * GBUF

    def start_gather(c):
        p = c % GBUF
        if sh[p] is not None:
            sh[p].wait()
            sh[p] = None
        gh[p] = pltpu.async_copy(
            xf_hbm.at[idx_v.at[pl.ds(c * CHG, CHG)]], bufs[p], sgs[p]
        )

    for c in range(min(GLOOK, n)):
        start_gather(c)
    for c in range(n):
        p = c % GBUF
        if c + GLOOK < n:
            start_gather(c + GLOOK)
        gh[p].wait()
        sh[p] = pltpu.async_copy(
            bufs[p], out_hbm.at[pl.ds(base + c * CHG, CHG)], sss[p]
        )
    for p in range(GBUF):
        if sh[p] is not None:
            sh[p].wait()


@functools.cache
def _gather():
    return pl.kernel(
        _gather_body,
        out_type=jax.ShapeDtypeStruct((P, DIM), jnp.float32),
        mesh=_sc_mesh(),
        scratch_types=(
            [pltpu.VMEM((RPW,), jnp.int32)]
            + [pltpu.VMEM((CHG, DIM), jnp.float32)] * GBUF
            + [pltpu.SemaphoreType.DMA] * (2 * GBUF)
        ),
    )


def _gmm_body(mp_ref, wgt_ref, xg_ref, w1_ref, w3_ref, w2_ref, out_ref):
    s = pl.program_id(0)
    jv = mp_ref[1, s]
    act = mp_ref[3, s] > 0

    @pl.when(act)
    def _():
        xb = xg_ref[...]
        a = jnp.dot(xb, w1_ref[0], preferred_element_type=jnp.float32)
        b = jnp.dot(xb, w3_ref[0], preferred_element_type=jnp.float32)
        h = ((a * jax.nn.sigmoid(a)) * b).astype(jnp.bfloat16)
        contrib = jnp.dot(h, w2_ref[0], preferred_element_type=jnp.float32)
        contrib = contrib * wgt_ref[0, 0, :][:, None]

        @pl.when(jv == 0)
        def _():
            out_ref[...] = contrib

        @pl.when(jv > 0)
        def _():
            out_ref[...] += contrib


def _gmm(mp, wgt_map, xg, w1, w2, w3):
    wgt3 = wgt_map.reshape(NB, 1, BLK)
    grid_spec = pltpu.PrefetchScalarGridSpec(
        num_scalar_prefetch=1,
        grid=(NB * NH,),
        in_specs=[
            pl.BlockSpec((1, 1, BLK), lambda s, m: (m[0, s], 0, 0)),
            pl.BlockSpec((BLK, DIM), lambda s, m: (m[0, s], 0)),
            pl.BlockSpec((1, DIM, BH), lambda s, m: (m[2, s], 0, m[1, s])),
            pl.BlockSpec((1, DIM, BH), lambda s, m: (m[2, s], 0, m[1, s])),
            pl.BlockSpec((1, BH, DIM), lambda s, m: (m[2, s], m[1, s], 0)),
        ],
        out_specs=pl.BlockSpec((BLK, DIM), lambda s, m: (m[0, s], 0)),
    )
    return pl.pallas_call(
        _gmm_body,
        grid_spec=grid_spec,
        out_shape=jax.ShapeDtypeStruct((P, DIM), jnp.float32),
        compiler_params=pltpu.CompilerParams(
            dimension_semantics=("arbitrary",),
        ),
    )(mp, wgt3, xg, w1, w3, w2)


def _combine_body(
    y_hbm, pos_hbm, out_hbm,
    i0, i1,
    r0a, r1a, oba, r0b, r1b, obb,
    g0a, g1a, g0b, g1b, sa, sb,
):
    wid = lax.axis_index("s") * NC + lax.axis_index("c")
    tb = wid * TPW
    pltpu.sync_copy(pos_hbm.at[0, pl.ds(tb, TPW)], i0)
    pltpu.sync_copy(pos_hbm.at[1, pl.ds(tb, TPW)], i1)
    r0s = (r0a, r0b)
    r1s = (r1a, r1b)
    obs = (oba, obb)
    g0s = (g0a, g0b)
    g1s = (g1a, g1b)
    sss = (sa, sb)
    n = TPW // CH2
    gh = [None, None]
    sh = [None, None]
    gh[0] = (
        pltpu.async_copy(y_hbm.at[i0.at[pl.ds(0, CH2)]], r0a, g0a),
        pltpu.async_copy(y_hbm.at[i1.at[pl.ds(0, CH2)]], r1a, g1a),
    )
    for c in range(n):
        p = c % 2
        q = (c + 1) % 2
        if c + 1 < n:
            gh[q] = (
                pltpu.async_copy(
                    y_hbm.at[i0.at[pl.ds((c + 1) * CH2, CH2)]], r0s[q], g0s[q]
                ),
                pltpu.async_copy(
                    y_hbm.at[i1.at[pl.ds((c + 1) * CH2, CH2)]], r1s[q], g1s[q]
                ),
            )
        gh[p][0].wait()
        gh[p][1].wait()
        if sh[p] is not None:
            sh[p].wait()
        r0 = r0s[p]
        r1 = r1s[p]
        ob = obs[p]

        @plsc.parallel_loop(0, CH2 * DIM // 16, 1, unroll=8)
        def _add(i):
            r = i // (DIM // 16)
            col = (i % (DIM // 16)) * 16
            ob[r, pl.ds(col, 16)] = r0[r, pl.ds(col, 16)] + r1[r, pl.ds(col, 16)]

        sh[p] = pltpu.async_copy(
            ob, out_hbm.at[pl.ds(tb + c * CH2, CH2)], sss[p]
        )
    for p in range(2):
        if sh[p] is not None:
            sh[p].wait()


@functools.cache
def _combine():
    return pl.kernel(
        _combine_body,
        out_type=jax.ShapeDtypeStruct((T, DIM), jnp.float32),
        mesh=_sc_mesh(),
        scratch_types=[
            pltpu.VMEM((TPW,), jnp.int32),
            pltpu.VMEM((TPW,), jnp.int32),
            pltpu.VMEM((CH2, DIM), jnp.float32),
            pltpu.VMEM((CH2, DIM), jnp.float32),
            pltpu.VMEM((CH2, DIM), jnp.float32),
            pltpu.VMEM((CH2, DIM), jnp.float32),
            pltpu.VMEM((CH2, DIM), jnp.float32),
            pltpu.VMEM((CH2, DIM), jnp.float32),
            pltpu.SemaphoreType.DMA,
            pltpu.SemaphoreType.DMA,
            pltpu.SemaphoreType.DMA,
            pltpu.SemaphoreType.DMA,
            pltpu.SemaphoreType.DMA,
            pltpu.SemaphoreType.DMA,
        ],
    )


def kernel(x, router_w, w1, w2, w3):
    bsz, seqlen, dim = x.shape
    xf = x.reshape(-1, dim)
    w_top, idx_top = _router(xf, router_w)
    tok_map, wgt_map, mp, pos_kt = _dispatch_maps(idx_top, w_top)
    xg = _gather()(xf, tok_map)
    y = _gmm(mp, wgt_map, xg, w1, w2, w3)
    out = _combine()(y, pos_kt)
    return out.reshape(bsz, seqlen, dim)
